# baseline (speedup 1.0000x reference)
import functools

import jax
import jax.numpy as jnp
from jax import lax
from jax.experimental import pallas as pl
from jax.experimental.pallas import tpu as pltpu
from jax.experimental.pallas import tpu_sc as plsc

N = 425984
D = 32
V = 1000001
VPAD = 1000008
NC = 2
NS = 16
CH_PER_SC = D // NC
B_PER_T = N // NS
CHUNK = 6656
NCHUNK = B_PER_T // CHUNK
STG = 62504
STG_LAST = VPAD - 15 * STG


@functools.partial(
    pl.kernel,
    out_type=jax.ShapeDtypeStruct((D * N,), jnp.float32),
    mesh=plsc.VectorSubcoreMesh(core_axis_name="c", subcore_axis_name="s"),
    scratch_types=[
        pltpu.VMEM_SHARED((VPAD,), jnp.float32),
        pltpu.VMEM((2, CHUNK), jnp.int32),
        pltpu.VMEM((B_PER_T,), jnp.float32),
        pltpu.SemaphoreType.DMA,
        pltpu.SemaphoreType.DMA((2,)),
        pltpu.SemaphoreType.DMA,
        pltpu.SemaphoreType.DMA,
    ],
    compiler_params=pltpu.CompilerParams(use_tc_tiling_on_sc=False),
)
def _bi_embed(kidx_hbm, vidx_hbm, kflat, vflat, out_flat,
              sp, idxc, outb, sems, semi, semg, semw):
    cid = lax.axis_index("c")
    sid = lax.axis_index("s")
    base = pl.multiple_of(sid * B_PER_T, 8)

    def stage(tab, c):
        off = pl.multiple_of((c * NC + cid) * VPAD + sid * STG, 8)
        dst_off = pl.multiple_of(sid * STG, 8)

        @pl.when(sid < NS - 1)
        def _():
            pltpu.async_copy(tab.at[pl.ds(off, STG)],
                             sp.at[pl.ds(dst_off, STG)], sems)

        @pl.when(sid == NS - 1)
        def _():
            pltpu.async_copy(tab.at[pl.ds(off, STG_LAST)],
                             sp.at[pl.ds(dst_off, STG_LAST)], sems)

    def stage_wait():
        @pl.when(sid < NS - 1)
        def _():
            pltpu.make_async_copy(
                kflat.at[pl.ds(0, STG)], sp.at[pl.ds(0, STG)], sems).wait()

        @pl.when(sid == NS - 1)
        def _():
            pltpu.make_async_copy(
                kflat.at[pl.ds(0, STG_LAST)], sp.at[pl.ds(0, STG_LAST)],
                sems).wait()

    def idx_load(src_hbm, j, b):
        pltpu.async_copy(
            src_hbm.at[pl.ds(base + j * CHUNK, CHUNK)], idxc.at[b], semi.at[b])

    def idx_wait(src_hbm, j, b):
        pltpu.make_async_copy(
            src_hbm.at[pl.ds(base + j * CHUNK, CHUNK)], idxc.at[b],
            semi.at[b]).wait()

    def phase(tab, src_idx_hbm, c, add):
        stage(tab, c)
        idx_load(src_idx_hbm, 0, 0)
        stage_wait()
        plsc.subcore_barrier()
        for j in range(NCHUNK):
            if j + 1 < NCHUNK:
                idx_load(src_idx_hbm, j + 1, (j + 1) % 2)
            idx_wait(src_idx_hbm, j, j % 2)
            dst = outb.at[pl.ds(j * CHUNK, CHUNK)]
            if add:
                pltpu.async_copy(sp.at[idxc.at[j % 2]], dst, semg, add=True)
            else:
                pltpu.async_copy(sp.at[idxc.at[j % 2]], dst, semg)
            pltpu.make_async_copy(sp.at[idxc.at[j % 2]], dst, semg).wait()
        plsc.subcore_barrier()

    def wb(c):
        off = pl.multiple_of((c * NC + cid) * N + base, 8)
        return pltpu.make_async_copy(
            outb, out_flat.at[pl.ds(off, B_PER_T)], semw)

    for c in range(CH_PER_SC):
        if c > 0:
            wb(c - 1).wait()
        phase(kflat, kidx_hbm, c, add=False)
        phase(vflat, vidx_hbm, c, add=True)
        wb(c).start()

    wb(CH_PER_SC - 1).wait()


def kernel(data, keys_table, values_table):
    kidx = data[:, 0]
    vidx = data[:, 1]
    kflat = jnp.pad(keys_table.T, ((0, 0), (0, VPAD - V))).reshape(-1)
    vflat = jnp.pad(values_table.T, ((0, 0), (0, VPAD - V))).reshape(-1)
    out_flat = _bi_embed(kidx, vidx, kflat, vflat)
    return out_flat.reshape(D, N).T


# speedup vs baseline: 4.9556x; 4.9556x over previous
import functools

import jax
import jax.numpy as jnp
from jax import lax
from jax.experimental import pallas as pl
from jax.experimental.pallas import tpu as pltpu
from jax.experimental.pallas import tpu_sc as plsc

N = 425984
D = 32
NC = 2
NS = 16
NW = NC * NS
B_PER_W = N // NW
CHUNK = 832
NCHUNK = B_PER_W // CHUNK
NBUF = 3


@functools.partial(
    pl.kernel,
    out_type=jax.ShapeDtypeStruct((N, D), jnp.float32),
    mesh=plsc.VectorSubcoreMesh(core_axis_name="c", subcore_axis_name="s"),
    scratch_types=[
        pltpu.VMEM((B_PER_W,), jnp.int32),
        pltpu.VMEM((B_PER_W,), jnp.int32),
        pltpu.VMEM((NBUF, CHUNK, D), jnp.float32),
        pltpu.SemaphoreType.DMA((NBUF,)),
        pltpu.SemaphoreType.DMA((NBUF,)),
        pltpu.SemaphoreType.DMA((NBUF,)),
    ],
    compiler_params=pltpu.CompilerParams(use_tc_tiling_on_sc=False),
)
def _bi_embed(kidx_hbm, vidx_hbm, keys_hbm, values_hbm, out_hbm,
              kidx_v, vidx_v, rows_v, semk, semv, semo):
    wid = lax.axis_index("s") * NC + lax.axis_index("c")
    base = pl.multiple_of(wid * B_PER_W, CHUNK)

    pltpu.sync_copy(kidx_hbm.at[pl.ds(base, B_PER_W)], kidx_v)
    pltpu.sync_copy(vidx_hbm.at[pl.ds(base, B_PER_W)], vidx_v)

    def gather_k(ci, b):
        src = keys_hbm.at[kidx_v.at[pl.ds(ci * CHUNK, CHUNK)]]
        return pltpu.make_async_copy(src, rows_v.at[b], semk.at[b])

    def gather_v_start(ci, b):
        src = values_hbm.at[vidx_v.at[pl.ds(ci * CHUNK, CHUNK)]]
        pltpu.async_copy(src, rows_v.at[b], semv.at[b], add=True)

    def gather_v_wait(ci, b):
        src = values_hbm.at[vidx_v.at[pl.ds(ci * CHUNK, CHUNK)]]
        pltpu.make_async_copy(src, rows_v.at[b], semv.at[b]).wait()

    def writeback(ci, b):
        off = pl.multiple_of(base + ci * CHUNK, CHUNK)
        return pltpu.make_async_copy(
            rows_v.at[b], out_hbm.at[pl.ds(off, CHUNK)], semo.at[b])

    def body(i, _):
        b0 = lax.rem(i, NBUF)
        j = i - 1
        k = i - 2
        b1 = lax.rem(j + NBUF, NBUF)
        b2 = lax.rem(k + NBUF, NBUF)

        @pl.when(i >= NBUF)
        def _():
            writeback(i - NBUF, b0).wait()

        @pl.when(i < NCHUNK)
        def _():
            gather_k(i, b0).start()

        @pl.when(jnp.logical_and(j >= 0, j < NCHUNK))
        def _():
            gather_k(j, b1).wait()
            gather_v_start(j, b1)

        @pl.when(jnp.logical_and(k >= 0, k < NCHUNK))
        def _():
            gather_v_wait(k, b2)
            writeback(k, b2).start()

        return 0

    lax.fori_loop(0, NCHUNK + 2, body, 0)

    writeback(NCHUNK - 1, (NCHUNK - 1) % NBUF).wait()


def kernel(data, keys_table, values_table):
    kidx = data[:, 0]
    vidx = data[:, 1]
    keys_rm = jax.lax.optimization_barrier(keys_table.T).T
    values_rm = jax.lax.optimization_barrier(values_table.T).T
    out = _bi_embed(kidx, vidx, keys_rm, values_rm)
    return jax.lax.optimization_barrier(out.T).T


# speedup vs baseline: 4.9562x; 1.0001x over previous
import functools

import jax
import jax.numpy as jnp
from jax import lax
from jax.experimental import pallas as pl
from jax.experimental.pallas import tpu as pltpu
from jax.experimental.pallas import tpu_sc as plsc

N = 425984
D = 32
NC = 2
NS = 16
NW = NC * NS
B_PER_W = N // NW
CHUNK = 832
NCHUNK = B_PER_W // CHUNK
NBUF = 3


@functools.partial(
    pl.kernel,
    out_type=jax.ShapeDtypeStruct((N, D), jnp.float32),
    mesh=plsc.VectorSubcoreMesh(core_axis_name="c", subcore_axis_name="s"),
    scratch_types=[
        pltpu.VMEM((B_PER_W,), jnp.int32),
        pltpu.VMEM((B_PER_W,), jnp.int32),
        pltpu.VMEM((NBUF, CHUNK, D), jnp.float32),
        pltpu.SemaphoreType.DMA((NBUF,)),
        pltpu.SemaphoreType.DMA((NBUF,)),
        pltpu.SemaphoreType.DMA((NBUF,)),
    ],
    compiler_params=pltpu.CompilerParams(use_tc_tiling_on_sc=False),
)
def _bi_embed(kidx_hbm, vidx_hbm, keys_hbm, values_hbm, out_hbm,
              kidx_v, vidx_v, rows_v, semk, semv, semo):
    wid = lax.axis_index("s") * NC + lax.axis_index("c")
    base = pl.multiple_of(wid * B_PER_W, CHUNK)

    pltpu.sync_copy(kidx_hbm.at[pl.ds(base, B_PER_W)], kidx_v)
    pltpu.sync_copy(vidx_hbm.at[pl.ds(base, B_PER_W)], vidx_v)

    def gather_k(ci, b):
        src = keys_hbm.at[kidx_v.at[pl.ds(ci * CHUNK, CHUNK)]]
        return pltpu.make_async_copy(src, rows_v.at[b], semk.at[b])

    def gather_v_start(ci, b):
        src = values_hbm.at[vidx_v.at[pl.ds(ci * CHUNK, CHUNK)]]
        pltpu.async_copy(src, rows_v.at[b], semv.at[b], add=True)

    def gather_v_wait(ci, b):
        src = values_hbm.at[vidx_v.at[pl.ds(ci * CHUNK, CHUNK)]]
        pltpu.make_async_copy(src, rows_v.at[b], semv.at[b]).wait()

    def writeback(ci, b):
        off = pl.multiple_of(base + ci * CHUNK, CHUNK)
        return pltpu.make_async_copy(
            rows_v.at[b], out_hbm.at[pl.ds(off, CHUNK)], semo.at[b])

    def body(i, _):
        b0 = lax.rem(i, NBUF)
        j = i - 1
        k = i - 2
        b1 = lax.rem(j + NBUF, NBUF)
        b2 = lax.rem(k + NBUF, NBUF)

        @pl.when(i >= NBUF)
        def _():
            writeback(i - NBUF, b0).wait()

        @pl.when(i < NCHUNK)
        def _():
            gather_k(i, b0).start()

        @pl.when(jnp.logical_and(j >= 0, j < NCHUNK))
        def _():
            gather_k(j, b1).wait()
            gather_v_start(j, b1)

        @pl.when(jnp.logical_and(k >= 0, k < NCHUNK))
        def _():
            gather_v_wait(k, b2)
            writeback(k, b2).start()

        return 0

    lax.fori_loop(0, NCHUNK + 2, body, 0)

    writeback(NCHUNK - 1, (NCHUNK - 1) % NBUF).wait()


def kernel(data, keys_table, values_table):
    kidx = data[:, 0]
    vidx = data[:, 1]
    return _bi_embed(kidx, vidx, keys_table, values_table)
